# baseline (speedup 1.0000x reference)
import jax
import jax.numpy as jnp
from jax import lax
from jax.experimental import pallas as pl
from jax.experimental.pallas import tpu as pltpu

P = 128
B = 1024
H = 128
ENC = 256
BB = 8

_f32 = jnp.float32
_bf16 = jnp.bfloat16


def _bdot(a, b):
    return jnp.dot(a.astype(_bf16), b.astype(_bf16),
                   preferred_element_type=_f32)


def _rowsum(x):
    acc = x[..., 0:8]
    for j in range(1, 16):
        acc = acc + x[..., 8 * j:8 * j + 8]
    acc = acc[..., 0:4] + acc[..., 4:8]
    acc = acc[..., 0:2] + acc[..., 2:4]
    return acc[..., 0:1] + acc[..., 1:2]


def _foldsum(x):
    w = x.shape[-1]
    while w > 1:
        h = w // 2
        x = x[..., :h] + x[..., h:w]
        w = h
    return x


def _body(coords_ref, flows_ref, Wc1T_ref, bc1_ref, Wc2T_ref, bc2_ref,
          Wc3T_ref, bc3_ref, Wg1_ref, bg1_ref, Wg2_ref, bg2_ref, Wg3_ref,
          bg3_ref, w2_u_ref, w1e_u_ref, w2_l_ref, w1e_l_ref,
          idx_u_ref, idx_l_ref, lp_u_ref, lp_l_ref):
    xc = coords_ref[...].reshape(BB * P, 2).astype(_bf16).astype(_f32)
    w1 = Wc1T_ref[...].astype(_bf16).astype(_f32)
    e1 = xc[:, 0:1] * w1[0:1, :] + xc[:, 1:2] * w1[1:2, :]
    e1 = e1 + bc1_ref[...]
    e2 = _bdot(e1, Wc2T_ref[...]) + bc2_ref[...]
    emb_c = (_bdot(e2, Wc3T_ref[...]) + bc3_ref[...]).reshape(BB, P, H)

    r = lax.broadcasted_iota(jnp.int32, (P, P), 0)
    c = lax.broadcasted_iota(jnp.int32, (P, P), 1)
    eye = (r == c).astype(_f32)
    bg1 = bg1_ref[...]
    bg2 = bg2_ref[...]
    bg3 = bg3_ref[...]
    emb_f_items = []
    for i in range(BB):
        F = flows_ref[i]
        A = F + eye
        deg = _rowsum(A)[:, 0]
        dinv = jnp.where(deg > 0, deg ** -0.5, 0.0)
        Anorm = dinv[:, None] * A * dinv[None, :]
        x1 = _bdot(Anorm, _bdot(F, Wg1_ref[...])) + bg1
        x2 = _bdot(Anorm, _bdot(x1, Wg2_ref[...])) + bg2
        x3 = _bdot(Anorm, _bdot(x2, Wg3_ref[...])) + bg3
        emb_f_items.append(x3)
    emb_f = jnp.stack(emb_f_items, axis=0)

    def pointer_pi(emb, w1e, w2):
        emb2d = emb.reshape(BB * P, H)
        la = _bdot(emb2d, w1e).reshape(BB, P)
        mx = jnp.max(la, axis=1, keepdims=True)
        ea = jnp.exp(la - mx)
        a = ea / _foldsum(ea)
        ctx_items = [_bdot(a[i:i + 1, :], emb[i]) for i in range(BB)]
        ctx = jnp.concatenate(ctx_items, axis=0)
        ctb = jnp.broadcast_to(ctx[:, None, :], (BB, P, H))
        cat2 = jnp.concatenate([emb, ctb], axis=-1).reshape(BB * P, 2 * H)
        return _bdot(cat2, w2).reshape(BB, P)

    pi_u = pointer_pi(emb_c, w1e_u_ref[...], w2_u_ref[...])
    pi_l = pointer_pi(emb_f, w1e_l_ref[...], w2_l_ref[...])

    lane = lax.broadcasted_iota(jnp.int32, (BB, P), 1)

    def decode(pi):
        def step(t, carry):
            mask, idx_acc, prob_acc = carry
            masked = pi + jnp.where(mask > 0, 0.0, -1e30)
            mx = jnp.max(masked, axis=1, keepdims=True)
            e = jnp.exp(masked - mx)
            p = e / _foldsum(e)
            pm = jnp.max(p, axis=1, keepdims=True)
            idx = jnp.min(jnp.where(p == pm, lane, P), axis=1, keepdims=True)
            sel = lane == t
            idx_acc = jnp.where(sel, idx.astype(_f32), idx_acc)
            prob_acc = jnp.where(sel, pm, prob_acc)
            mask = jnp.where(lane == idx, 0.0, mask)
            return mask, idx_acc, prob_acc

        init = (jnp.ones((BB, P), _f32), jnp.zeros((BB, P), _f32),
                jnp.ones((BB, P), _f32))
        _, idx_acc, prob_acc = lax.fori_loop(0, P, step, init)
        return idx_acc.astype(jnp.int32), jnp.log(prob_acc)

    idx_u_ref[...], lp_u_ref[...] = decode(pi_u)
    idx_l_ref[...], lp_l_ref[...] = decode(pi_l)


def kernel(coords, flows, Wc1, bc1, Wc2, bc2, Wc3, bc3, Wg1, bg1, Wg2, bg2,
           Wg3, bg3, Wih_u, Whh_u, bih_u, bhh_u, Wih_l, Whh_l, bih_l, bhh_l,
           pt1_u, pt2_u, pt1_l, pt2_l):
    grid = (B // BB,)
    full = lambda arr: pl.BlockSpec(arr.shape, lambda i: (0,) * arr.ndim)

    weights = [Wc1.T, bc1.reshape(1, H), Wc2.T, bc2.reshape(1, ENC),
               Wc3.T, bc3.reshape(1, H), Wg1, bg1.reshape(1, H),
               Wg2, bg2.reshape(1, ENC), Wg3, bg3.reshape(1, H),
               pt2_u.reshape(2 * H, 1), pt1_u[:H].reshape(H, 1),
               pt2_l.reshape(2 * H, 1), pt1_l[:H].reshape(H, 1)]

    in_specs = [
        pl.BlockSpec((BB, P, 2), lambda i: (i, 0, 0)),
        pl.BlockSpec((BB, P, P), lambda i: (i, 0, 0)),
    ] + [full(w) for w in weights]

    out_shape = [
        jax.ShapeDtypeStruct((B, P), jnp.int32),
        jax.ShapeDtypeStruct((B, P), jnp.int32),
        jax.ShapeDtypeStruct((B, P), _f32),
        jax.ShapeDtypeStruct((B, P), _f32),
    ]
    out_specs = [pl.BlockSpec((BB, P), lambda i: (i, 0))] * 4

    idx_u, idx_l, lp_u, lp_l = pl.pallas_call(
        _body,
        grid=grid,
        in_specs=in_specs,
        out_specs=out_specs,
        out_shape=out_shape,
        compiler_params=pltpu.CompilerParams(
            dimension_semantics=("parallel",)),
    )(coords, flows, *weights)

    indices = jnp.stack([idx_u.T, idx_l.T], axis=-1)
    log_probs = jnp.stack([lp_u.T, lp_l.T], axis=-1)
    return indices, log_probs
